# baseline (speedup 1.0000x reference)
import functools

import jax
import jax.numpy as jnp
from jax import lax
from jax.experimental import pallas as pl

_EPS = 1e-12


def _body(grid_ref, row_ref, col_ref, tok_ref, lnw_ref, lnb_ref, out_ref):
    x = grid_ref[0]
    f = x.shape[0]
    m = jnp.sum(x, axis=0) * (1.0 / f)
    row = row_ref[0]
    col = col_ref[...]
    tok = tok_ref[...]
    bias = row[:, None, :] + (col + tok)[None, :, :]
    e = m + bias
    mu = jnp.mean(e, axis=-1, keepdims=True)
    d = e - mu
    var = jnp.mean(d * d, axis=-1, keepdims=True)
    inv = lax.rsqrt(var + _EPS)
    out_ref[0] = (d * inv * lnw_ref[...][None, :, :]
                  + lnb_ref[...][None, :, :])


@functools.partial(jax.jit, static_argnames=("rb",))
def _fused(grid, row_emb, col_emb, tok_emb, ln_w, ln_b, rb=4):
    B, F, H, W, C = grid.shape
    out = pl.pallas_call(
        _body,
        grid=(B, H // rb),
        in_specs=[
            pl.BlockSpec((1, F, rb, W, C), lambda b, j: (b, 0, j, 0, 0)),
            pl.BlockSpec((1, rb, C), lambda b, j: (j, 0, 0)),
            pl.BlockSpec((W, C), lambda b, j: (0, 0)),
            pl.BlockSpec((1, C), lambda b, j: (0, 0)),
            pl.BlockSpec((1, C), lambda b, j: (0, 0)),
            pl.BlockSpec((1, C), lambda b, j: (0, 0)),
        ],
        out_specs=pl.BlockSpec((1, rb, W, C), lambda b, j: (b, j, 0, 0)),
        out_shape=jax.ShapeDtypeStruct((B, H, W, C), grid.dtype),
    )(grid, row_emb[:H].reshape(H // rb, rb, C), col_emb[:W],
      tok_emb.reshape(1, C),
      ln_w.reshape(1, C), ln_b.reshape(1, C))

    return out.reshape(B, H * W, C)


def kernel(grid, row_emb, col_emb, tok_emb, ln_w, ln_b):
    return _fused(grid, row_emb, col_emb, tok_emb, ln_w, ln_b)


# speedup vs baseline: 1.3796x; 1.3796x over previous
import functools

import jax
import jax.numpy as jnp
from jax import lax
from jax.experimental import pallas as pl

_EPS = 1e-12


def _body(grid_ref, row_ref, col_ref, tok_ref, lnw_ref, lnb_ref, out_ref):
    x = grid_ref[0]
    f = x.shape[0]
    m = jnp.sum(x, axis=0) * (1.0 / f)
    row = row_ref[0]
    col = col_ref[...]
    tok = tok_ref[...]
    bias = row[:, None, :] + (col + tok)[None, :, :]
    e = m + bias
    mu = jnp.mean(e, axis=-1, keepdims=True)
    d = e - mu
    var = jnp.mean(d * d, axis=-1, keepdims=True)
    inv = lax.rsqrt(var + _EPS)
    out_ref[0] = (d * inv * lnw_ref[...][None, :, :]
                  + lnb_ref[...][None, :, :])


@functools.partial(jax.jit, static_argnames=("rb",))
def _fused(grid, row_emb, col_emb, tok_emb, ln_w, ln_b, rb=24):
    B, F, H, W, C = grid.shape
    out = pl.pallas_call(
        _body,
        grid=(B, H // rb),
        in_specs=[
            pl.BlockSpec((1, F, rb, W, C), lambda b, j: (b, 0, j, 0, 0)),
            pl.BlockSpec((1, rb, C), lambda b, j: (j, 0, 0)),
            pl.BlockSpec((W, C), lambda b, j: (0, 0)),
            pl.BlockSpec((1, C), lambda b, j: (0, 0)),
            pl.BlockSpec((1, C), lambda b, j: (0, 0)),
            pl.BlockSpec((1, C), lambda b, j: (0, 0)),
        ],
        out_specs=pl.BlockSpec((1, rb, W, C), lambda b, j: (b, j, 0, 0)),
        out_shape=jax.ShapeDtypeStruct((B, H, W, C), grid.dtype),
    )(grid, row_emb[:H].reshape(H // rb, rb, C), col_emb[:W],
      tok_emb.reshape(1, C),
      ln_w.reshape(1, C), ln_b.reshape(1, C))

    return out.reshape(B, H * W, C)


def kernel(grid, row_emb, col_emb, tok_emb, ln_w, ln_b):
    return _fused(grid, row_emb, col_emb, tok_emb, ln_w, ln_b)
